# baseline (speedup 1.0000x reference)
import functools

import jax
import jax.numpy as jnp
from jax import lax
from jax.experimental import pallas as pl
from jax.experimental.pallas import tpu as pltpu
from jax.experimental.pallas import tpu_sc as plsc

D = 64
L = 199
NW = 32
CH = 128


def _resppos_body(resp_ref, pos_ref, out_ref):
    out_ref[...] = resp_ref[...][:, None, :] + pos_ref[...][None, :, :]


def _build_resppos(w_resp, w_pos):
    out = pl.pallas_call(
        _resppos_body,
        out_shape=jax.ShapeDtypeStruct((4, L, D), jnp.float32),
    )(w_resp, w_pos)
    return out.reshape(4 * L, D)


def _sc_body(idx_hbm, w_exe, w_cat, w_tag, w_rp, out_hbm,
             idx_v0, idx_v1, idxrp_v0, idxrp_v1,
             be0, bc0, bt0, brp0, be1, bc1, bt1, brp1, ob0, ob1,
             sg0, sg1, si0, si1, sw0, sw1):
    wid = lax.axis_index("s") * 2 + lax.axis_index("c")
    b0 = wid * CH
    npair = (L - 1) // 2

    sets = [
        (idx_v0, idxrp_v0, be0, bc0, bt0, brp0, sg0, si0, ob0, sw0),
        (idx_v1, idxrp_v1, be1, bc1, bt1, brp1, sg1, si1, ob1, sw1),
    ]

    def idx_desc(l, st):
        return pltpu.make_async_copy(
            idx_hbm.at[:, l, pl.ds(b0, CH)], st[0], st[7])

    def compute_rp(l, st):
        idx_v, idxrp_v = st[0], st[1]
        for s in range(CH // 16):
            rv = idx_v[3, pl.ds(s * 16, 16)]
            idxrp_v[pl.ds(s * 16, 16)] = rv * L + l

    def gather_descs(st):
        idx_v, idxrp_v, be, bc, bt, brp, sg = st[:7]
        return (pltpu.make_async_copy(w_exe.at[idx_v.at[0]], be, sg),
                pltpu.make_async_copy(w_cat.at[idx_v.at[1]], bc, sg),
                pltpu.make_async_copy(w_tag.at[idx_v.at[2]], bt, sg),
                pltpu.make_async_copy(w_rp.at[idxrp_v], brp, sg))

    def fire_gathers(st):
        for d in gather_descs(st):
            d.start()

    def wait_gathers(st):
        for d in gather_descs(st):
            d.wait()

    def wb_desc(l, st):
        return pltpu.make_async_copy(st[8], out_hbm.at[l, :, wid], st[9])

    def reduce(st):
        be, bc, bt, brp = st[2], st[3], st[4], st[5]
        ob = st[8]
        iota = lax.iota(jnp.int32, 16)
        rows = [iota + (g * 16) for g in range(CH // 16)]

        def red(d, _):
            dv = jnp.full((16,), jnp.int32(0)) + d
            t = d >> 3
            off = (d & 7) * 128
            for g in range(CH // 16):
                e = plsc.load_gather(be, [rows[g], dv])
                c = plsc.load_gather(bc, [rows[g], dv])
                tt = plsc.load_gather(bt, [rows[g], dv])
                rp = plsc.load_gather(brp, [rows[g], dv])
                ob[t, pl.ds(off + g * 16, 16)] = (e + c) + (tt + rp)
            return _

        lax.fori_loop(0, D, red, None)

    pltpu.sync_copy(idx_hbm.at[:, 0, pl.ds(b0, CH)], idx_v0)
    compute_rp(0, sets[0])
    fire_gathers(sets[0])
    idx_desc(1, sets[1]).start()

    def pair(p, _):
        lb = 2 * p + 1
        lc = 2 * p + 2
        idx_desc(lb, sets[1]).wait()
        compute_rp(lb, sets[1])
        fire_gathers(sets[1])
        wait_gathers(sets[0])
        idx_desc(lc, sets[0]).start()
        pl.when(p > 0)(lambda: wb_desc(2 * p - 2, sets[0]).wait())
        reduce(sets[0])
        wb_desc(2 * p, sets[0]).start()
        idx_desc(lc, sets[0]).wait()
        compute_rp(lc, sets[0])
        fire_gathers(sets[0])
        wait_gathers(sets[1])
        pl.when(p < npair - 1)(lambda: idx_desc(2 * p + 3, sets[1]).start())
        pl.when(p > 0)(lambda: wb_desc(2 * p - 1, sets[1]).wait())
        reduce(sets[1])
        wb_desc(lb, sets[1]).start()
        return _

    lax.fori_loop(0, npair, pair, None)
    wait_gathers(sets[0])
    wb_desc(L - 3, sets[0]).wait()
    reduce(sets[0])
    wb_desc(L - 1, sets[0]).start()
    wb_desc(L - 1, sets[0]).wait()
    wb_desc(L - 2, sets[1]).wait()


def kernel(input_e, category, tag, chapter, test, response,
           W_exe, W_cat, W_tag, W_chap, W_test, W_pos, W_resp):
    del chapter, test, W_chap, W_test
    B, Lc = input_e.shape
    n = B * Lc

    idx_t = jnp.stack([
        input_e.T.astype(jnp.int32),
        category.T.astype(jnp.int32),
        tag.T.astype(jnp.int32),
        response.T.astype(jnp.int32),
    ])
    w_rp = _build_resppos(W_resp.astype(jnp.float32),
                          W_pos[:L].astype(jnp.float32))

    mesh = plsc.VectorSubcoreMesh(core_axis_name="c", subcore_axis_name="s")
    out = pl.kernel(
        _sc_body,
        out_type=jax.ShapeDtypeStruct((Lc, D // 8, NW, 8 * CH), jnp.float32),
        mesh=mesh,
        compiler_params=pltpu.CompilerParams(use_tc_tiling_on_sc=False,
                                             needs_layout_passes=False),
        scratch_types=(
            [pltpu.VMEM((4, CH), jnp.int32)] * 2
            + [pltpu.VMEM((CH,), jnp.int32)] * 2
            + [pltpu.VMEM((CH, D), jnp.float32)] * 8
            + [pltpu.VMEM((D // 8, 8 * CH), jnp.float32)] * 2
            + [pltpu.SemaphoreType.DMA] * 6
        ),
    )(idx_t, W_exe, W_cat, W_tag, w_rp)
    out = out.reshape(Lc, D // 8, NW, 8, CH)
    out = out.transpose(2, 4, 0, 1, 3).reshape(B, Lc, D)
    return out


# speedup vs baseline: 3.4863x; 3.4863x over previous
import functools

import jax
import jax.numpy as jnp
from jax import lax
from jax.experimental import pallas as pl
from jax.experimental.pallas import tpu as pltpu
from jax.experimental.pallas import tpu_sc as plsc

D = 64
L = 199
NW = 32
CH = 128


def _resppos_body(resp_ref, pos_ref, out_ref):
    out_ref[...] = resp_ref[...][:, None, :] + pos_ref[...][None, :, :]


def _build_resppos(w_resp, w_pos):
    out = pl.pallas_call(
        _resppos_body,
        out_shape=jax.ShapeDtypeStruct((4, L, D), jnp.float32),
    )(w_resp, w_pos)
    return out.reshape(4 * L, D)


def _sc_body(idx_hbm, w_exe, w_cat, w_tag, w_rp, out_hbm,
             idx_v0, idx_v1, idxrp_v0, idxrp_v1,
             be0, bc0, bt0, brp0, be1, bc1, bt1, brp1, ob0, ob1,
             sg0, sg1, si0, si1, sw0, sw1):
    wid = lax.axis_index("s") * 2 + lax.axis_index("c")
    b0 = wid * CH
    npair = (L - 1) // 2

    sets = [
        (idx_v0, idxrp_v0, be0, bc0, bt0, brp0, sg0, si0, ob0, sw0),
        (idx_v1, idxrp_v1, be1, bc1, bt1, brp1, sg1, si1, ob1, sw1),
    ]

    def idx_desc(l, st):
        return pltpu.make_async_copy(
            idx_hbm.at[:, l, pl.ds(b0, CH)], st[0], st[7])

    def compute_rp(l, st):
        idx_v, idxrp_v = st[0], st[1]
        for s in range(CH // 16):
            rv = idx_v[3, pl.ds(s * 16, 16)]
            idxrp_v[pl.ds(s * 16, 16)] = rv * L + l

    def gather_descs(st):
        idx_v, idxrp_v, be, bc, bt, brp, sg = st[:7]
        return (pltpu.make_async_copy(w_exe.at[idx_v.at[0]], be, sg),
                pltpu.make_async_copy(w_cat.at[idx_v.at[1]], bc, sg),
                pltpu.make_async_copy(w_tag.at[idx_v.at[2]], bt, sg),
                pltpu.make_async_copy(w_rp.at[idxrp_v], brp, sg))

    def fire_gathers(st):
        for d in gather_descs(st):
            d.start()

    def wait_gathers(st):
        for d in gather_descs(st):
            d.wait()

    def wb_desc(l, st):
        return pltpu.make_async_copy(
            st[8].at[:, :, pl.ds(0, CH)], out_hbm.at[l, :, wid], st[9])

    def reduce(st):
        be, bc, bt, brp = st[2], st[3], st[4], st[5]
        ob = st[8]
        iota = lax.iota(jnp.int32, 16)
        dvs = [iota + (s * 16) for s in range(D // 16)]
        tvs = [dv >> 3 for dv in dvs]
        divs = [dv & 7 for dv in dvs]

        def red(b, _):
            bsplat = jnp.full((16,), b, jnp.int32)
            for s in range(D // 16):
                sl = pl.ds(s * 16, 16)
                v = (be[b, sl] + bc[b, sl]) + (bt[b, sl] + brp[b, sl])
                plsc.store_scatter(ob, [tvs[s], divs[s], bsplat], v)
            return _

        lax.fori_loop(0, CH, red, None)

    pltpu.sync_copy(idx_hbm.at[:, 0, pl.ds(b0, CH)], idx_v0)
    compute_rp(0, sets[0])
    fire_gathers(sets[0])
    idx_desc(1, sets[1]).start()

    def pair(p, _):
        lb = 2 * p + 1
        lc = 2 * p + 2
        idx_desc(lb, sets[1]).wait()
        compute_rp(lb, sets[1])
        fire_gathers(sets[1])
        wait_gathers(sets[0])
        idx_desc(lc, sets[0]).start()
        pl.when(p > 0)(lambda: wb_desc(2 * p - 2, sets[0]).wait())
        reduce(sets[0])
        wb_desc(2 * p, sets[0]).start()
        idx_desc(lc, sets[0]).wait()
        compute_rp(lc, sets[0])
        fire_gathers(sets[0])
        wait_gathers(sets[1])
        pl.when(p < npair - 1)(lambda: idx_desc(2 * p + 3, sets[1]).start())
        pl.when(p > 0)(lambda: wb_desc(2 * p - 1, sets[1]).wait())
        reduce(sets[1])
        wb_desc(lb, sets[1]).start()
        return _

    lax.fori_loop(0, npair, pair, None)
    wait_gathers(sets[0])
    wb_desc(L - 3, sets[0]).wait()
    reduce(sets[0])
    wb_desc(L - 1, sets[0]).start()
    wb_desc(L - 1, sets[0]).wait()
    wb_desc(L - 2, sets[1]).wait()


def kernel(input_e, category, tag, chapter, test, response,
           W_exe, W_cat, W_tag, W_chap, W_test, W_pos, W_resp):
    del chapter, test, W_chap, W_test
    B, Lc = input_e.shape
    n = B * Lc

    idx_t = jnp.stack([
        input_e.T.astype(jnp.int32),
        category.T.astype(jnp.int32),
        tag.T.astype(jnp.int32),
        response.T.astype(jnp.int32),
    ])
    w_rp = _build_resppos(W_resp.astype(jnp.float32),
                          W_pos[:L].astype(jnp.float32))

    mesh = plsc.VectorSubcoreMesh(core_axis_name="c", subcore_axis_name="s")
    out = pl.kernel(
        _sc_body,
        out_type=jax.ShapeDtypeStruct((Lc, D // 8, NW, 8, CH), jnp.float32),
        mesh=mesh,
        compiler_params=pltpu.CompilerParams(use_tc_tiling_on_sc=False,
                                             needs_layout_passes=False),
        scratch_types=(
            [pltpu.VMEM((4, CH), jnp.int32)] * 2
            + [pltpu.VMEM((CH,), jnp.int32)] * 2
            + [pltpu.VMEM((CH, D), jnp.float32)] * 8
            + [pltpu.VMEM((D // 8, 8, CH + 1), jnp.float32)] * 2
            + [pltpu.SemaphoreType.DMA] * 6
        ),
    )(idx_t, W_exe, W_cat, W_tag, w_rp)
    return out.transpose(2, 4, 0, 1, 3).reshape(B, Lc, D)
